# baseline (speedup 1.0000x reference)
import functools

import jax
import jax.numpy as jnp
from jax import lax
from jax.experimental import pallas as pl
from jax.experimental.pallas import tpu as pltpu
from jax.experimental.pallas import tpu_sc as plsc

NUM_FIELDS = 26
VOCAB = 100000
EMB_DIM = 16
EPS = 1e-5
NGRP_F = 4
D_PAD = NGRP_F * 128


_TCHUNK = 16384


def _transpose_body(g, x_ref, o_ref):
    x = x_ref[...]
    if g == NGRP_F - 1:
        row = lax.broadcasted_iota(jnp.int32, x.shape, 0)
        x = jnp.where(row < (NUM_FIELDS * EMB_DIM - 128 * g), x, 0.0)
    o_ref[...] = jnp.transpose(x)


def _table_group(t416, g):
    ncol = (VOCAB + _TCHUNK - 1) // _TCHUNK
    return pl.pallas_call(
        functools.partial(_transpose_body, g),
        grid=(ncol,),
        in_specs=[pl.BlockSpec((128, _TCHUNK), lambda c: (g, c))],
        out_specs=pl.BlockSpec((_TCHUNK, 128), lambda c: (c, 0)),
        out_shape=jax.ShapeDtypeStruct((VOCAB, 128), jnp.float32),
    )(t416)


def _make_sc_gather(total_rows: int):
    num_cores, num_subcores = 2, 16
    nw = num_cores * num_subcores
    rows_per_w = total_rows // nw
    n_idx_rows = rows_per_w // 128
    GPER = 8
    NGRP = n_idx_rows // GPER
    GROWS = GPER * 128

    mesh = plsc.VectorSubcoreMesh(core_axis_name="c", subcore_axis_name="s",
                                  num_cores=num_cores,
                                  num_subcores=num_subcores)

    @functools.partial(
        pl.kernel,
        mesh=mesh,
        out_type=jax.ShapeDtypeStruct((total_rows, EMB_DIM), jnp.float32),
        scratch_types=[
            pltpu.VMEM((n_idx_rows, 128), jnp.int32),
            pltpu.VMEM((GROWS, EMB_DIM), jnp.float32),
            pltpu.SemaphoreType.DMA,
        ],
        compiler_params=pltpu.CompilerParams(use_tc_tiling_on_sc=False),
    )
    def gather_kernel(table_hbm, idx_hbm, out_hbm, idx_v, rows_v, sem):
        wid = lax.axis_index("s") * num_cores + lax.axis_index("c")
        idx_base = wid * n_idx_rows
        out_base = wid * rows_per_w
        pltpu.sync_copy(idx_hbm.at[pl.ds(idx_base, n_idx_rows)], idx_v)

        def group(g):
            handles = []
            for j in range(GPER):
                handles.append(pltpu.async_copy(
                    table_hbm.at[idx_v.at[g * GPER + j]],
                    rows_v.at[pl.ds(j * 128, 128)],
                    sem))
            for h in handles:
                h.wait()
            pltpu.sync_copy(rows_v, out_hbm.at[pl.ds(out_base + g * GROWS, GROWS)])

        lax.fori_loop(0, NGRP, lambda g, _: (group(g), 0)[1], 0)

    return gather_kernel


def _stats_body(nb, x0_ref, x1_ref, x2_ref, x3_ref, o_ref, acc):
    i = pl.program_id(0)

    @pl.when(i == 0)
    def _():
        acc[...] = jnp.zeros_like(acc)

    x = jnp.concatenate(
        [x0_ref[...], x1_ref[...], x2_ref[...], x3_ref[...]], axis=1)
    s = jnp.sum(x, axis=0, keepdims=True)
    q = jnp.sum(x * x, axis=0, keepdims=True)
    acc[...] += jnp.concatenate([s, q], axis=0)

    @pl.when(i == nb - 1)
    def _():
        o_ref[...] = acc[...]


def _column_stats4(xs, blk):
    b = xs[0].shape[0]
    nb = b // blk
    return pl.pallas_call(
        functools.partial(_stats_body, nb),
        grid=(nb,),
        in_specs=[pl.BlockSpec((blk, 128), lambda i: (i, 0))] * NGRP_F,
        out_specs=pl.BlockSpec((2, D_PAD), lambda i: (0, 0)),
        out_shape=jax.ShapeDtypeStruct((2, D_PAD), jnp.float32),
        scratch_shapes=[pltpu.VMEM((2, D_PAD), jnp.float32)],
    )(*xs)


def _layer1_body(nb, inv_b, x0_ref, x1_ref, x2_ref, x3_ref, st_ref, g_ref,
                 b_ref, w_ref, bias_ref, h_ref, ost_ref, acc):
    i = pl.program_id(0)
    mu = st_ref[0:1, :] * inv_b
    var = st_ref[1:2, :] * inv_b - mu * mu
    s = g_ref[...] * lax.rsqrt(var + EPS)
    t = b_ref[...] - mu * s
    x = jnp.concatenate(
        [x0_ref[...], x1_ref[...], x2_ref[...], x3_ref[...]], axis=1)
    xn = x * s + t
    h = jnp.dot(xn, w_ref[...], preferred_element_type=jnp.float32)
    h = jnp.maximum(h + bias_ref[...], 0.0)
    h_ref[...] = h

    @pl.when(i == 0)
    def _():
        acc[...] = jnp.zeros_like(acc)

    hs = jnp.sum(h, axis=0, keepdims=True)
    hq = jnp.sum(h * h, axis=0, keepdims=True)
    acc[...] += jnp.concatenate([hs, hq], axis=0)

    @pl.when(i == nb - 1)
    def _():
        ost_ref[...] = acc[...]


def _layer1(xs, stats, g, b, w, bias, blk):
    bsz = xs[0].shape[0]
    dout = w.shape[1]
    nb = bsz // blk
    return pl.pallas_call(
        functools.partial(_layer1_body, nb, 1.0 / bsz),
        grid=(nb,),
        in_specs=[pl.BlockSpec((blk, 128), lambda i: (i, 0))] * NGRP_F + [
            pl.BlockSpec((2, D_PAD), lambda i: (0, 0)),
            pl.BlockSpec((1, D_PAD), lambda i: (0, 0)),
            pl.BlockSpec((1, D_PAD), lambda i: (0, 0)),
            pl.BlockSpec((D_PAD, dout), lambda i: (0, 0)),
            pl.BlockSpec((1, dout), lambda i: (0, 0)),
        ],
        out_specs=[
            pl.BlockSpec((blk, dout), lambda i: (i, 0)),
            pl.BlockSpec((2, dout), lambda i: (0, 0)),
        ],
        out_shape=[
            jax.ShapeDtypeStruct((bsz, dout), jnp.float32),
            jax.ShapeDtypeStruct((2, dout), jnp.float32),
        ],
        scratch_shapes=[pltpu.VMEM((2, dout), jnp.float32)],
    )(*xs, stats, g, b, w, bias)


def _layer_body(nb, inv_b, x_ref, st_ref, g_ref, b_ref, w_ref, bias_ref,
                h_ref, ost_ref, acc):
    i = pl.program_id(0)
    mu = st_ref[0:1, :] * inv_b
    var = st_ref[1:2, :] * inv_b - mu * mu
    s = g_ref[...] * lax.rsqrt(var + EPS)
    t = b_ref[...] - mu * s
    xn = x_ref[...] * s + t
    h = jnp.dot(xn, w_ref[...], preferred_element_type=jnp.float32)
    h = jnp.maximum(h + bias_ref[...], 0.0)
    h_ref[...] = h

    @pl.when(i == 0)
    def _():
        acc[...] = jnp.zeros_like(acc)

    hs = jnp.sum(h, axis=0, keepdims=True)
    hq = jnp.sum(h * h, axis=0, keepdims=True)
    acc[...] += jnp.concatenate([hs, hq], axis=0)

    @pl.when(i == nb - 1)
    def _():
        ost_ref[...] = acc[...]


def _norm_layer(x, stats, g, b, w, bias, blk):
    bsz, din = x.shape
    dout = w.shape[1]
    nb = bsz // blk
    return pl.pallas_call(
        functools.partial(_layer_body, nb, 1.0 / bsz),
        grid=(nb,),
        in_specs=[
            pl.BlockSpec((blk, din), lambda i: (i, 0)),
            pl.BlockSpec((2, din), lambda i: (0, 0)),
            pl.BlockSpec((1, din), lambda i: (0, 0)),
            pl.BlockSpec((1, din), lambda i: (0, 0)),
            pl.BlockSpec((din, dout), lambda i: (0, 0)),
            pl.BlockSpec((1, dout), lambda i: (0, 0)),
        ],
        out_specs=[
            pl.BlockSpec((blk, dout), lambda i: (i, 0)),
            pl.BlockSpec((2, dout), lambda i: (0, 0)),
        ],
        out_shape=[
            jax.ShapeDtypeStruct((bsz, dout), jnp.float32),
            jax.ShapeDtypeStruct((2, dout), jnp.float32),
        ],
        scratch_shapes=[pltpu.VMEM((2, dout), jnp.float32)],
    )(x, stats, g.reshape(1, din), b.reshape(1, din), w, bias.reshape(1, dout))


def _final_body(inv_b, x_ref, st_ref, g_ref, b_ref, w_ref, bias_ref, o_ref):
    mu = st_ref[0:1, :] * inv_b
    var = st_ref[1:2, :] * inv_b - mu * mu
    s = g_ref[...] * lax.rsqrt(var + EPS)
    t = b_ref[...] - mu * s
    xn = x_ref[...] * s + t
    z = jnp.sum(xn * w_ref[...], axis=1, keepdims=True) + bias_ref[0, 0]
    o_ref[...] = jax.nn.sigmoid(z)


def _final_layer(x, stats, g, b, w3, b3, blk):
    bsz, din = x.shape
    nb = bsz // blk
    return pl.pallas_call(
        functools.partial(_final_body, 1.0 / bsz),
        grid=(nb,),
        in_specs=[
            pl.BlockSpec((blk, din), lambda i: (i, 0)),
            pl.BlockSpec((2, din), lambda i: (0, 0)),
            pl.BlockSpec((1, din), lambda i: (0, 0)),
            pl.BlockSpec((1, din), lambda i: (0, 0)),
            pl.BlockSpec((1, din), lambda i: (0, 0)),
            pl.BlockSpec((1, 1), lambda i: (0, 0)),
        ],
        out_specs=pl.BlockSpec((blk, 1), lambda i: (i, 0)),
        out_shape=jax.ShapeDtypeStruct((bsz, 1), jnp.float32),
    )(x, stats, g.reshape(1, din), b.reshape(1, din),
      w3.reshape(1, din), b3.reshape(1, 1))


def kernel(x_cat, tables, W1, b1, W2, b2, W3, b3,
           bn0_g, bn0_b, bn1_g, bn1_b, bn2_g, bn2_b):
    bsz = x_cat.shape[0]

    t416 = tables.transpose(0, 2, 1).reshape(NUM_FIELDS * EMB_DIM, VOCAB)

    xc = x_cat.astype(jnp.int32)
    xc_pad = jnp.concatenate(
        [xc, jnp.zeros((bsz, NGRP_F * 8 - NUM_FIELDS), jnp.int32)], axis=1)
    fl = jnp.arange(NGRP_F * 8, dtype=jnp.int32)[None, :] % 8
    idx_all = xc_pad * 8 + fl

    gather = _make_sc_gather(bsz * 8)
    xs = []
    for g in range(NGRP_F):
        tg = _table_group(t416, g).reshape(VOCAB * 8, EMB_DIM)
        idx_g = idx_all[:, 8 * g:8 * (g + 1)].reshape(bsz * 8 // 128, 128)
        rows = gather(tg, idx_g)
        xs.append(rows.reshape(bsz, 128))

    dpadc = D_PAD - NUM_FIELDS * EMB_DIM
    g0 = jnp.concatenate([bn0_g, jnp.zeros((dpadc,), jnp.float32)])
    b0 = jnp.concatenate([bn0_b, jnp.zeros((dpadc,), jnp.float32)])
    w1 = jnp.concatenate(
        [W1, jnp.zeros((dpadc, W1.shape[1]), jnp.float32)], axis=0)

    blk = 2048
    st0 = _column_stats4(xs, blk)
    h1, st1 = _layer1(xs, st0, g0.reshape(1, D_PAD), b0.reshape(1, D_PAD),
                      w1, b1.reshape(1, -1), blk)
    h2, st2 = _norm_layer(h1, st1, bn1_g, bn1_b, W2, b2, blk)
    out = _final_layer(h2, st2, bn2_g, bn2_b, W3, b3, blk)
    return out.reshape(bsz)


# speedup vs baseline: 2.6254x; 2.6254x over previous
import functools

import jax
import jax.numpy as jnp
from jax import lax
from jax.experimental import pallas as pl
from jax.experimental.pallas import tpu as pltpu
from jax.experimental.pallas import tpu_sc as plsc

NUM_FIELDS = 26
VOCAB = 100000
EMB_DIM = 16
EPS = 1e-5
NGRP_F = 4
D_PAD = NGRP_F * 128


_TCHUNK = 16384


def _transpose_body(g, x_ref, o_ref):
    x = x_ref[...]
    if g == NGRP_F - 1:
        row = lax.broadcasted_iota(jnp.int32, x.shape, 0)
        x = jnp.where(row < (NUM_FIELDS * EMB_DIM - 128 * g), x, 0.0)
    o_ref[...] = jnp.transpose(x)


def _table_group(t416, g):
    ncol = (VOCAB + _TCHUNK - 1) // _TCHUNK
    return pl.pallas_call(
        functools.partial(_transpose_body, g),
        grid=(ncol,),
        in_specs=[pl.BlockSpec((128, _TCHUNK), lambda c: (g, c))],
        out_specs=pl.BlockSpec((_TCHUNK, 128), lambda c: (c, 0)),
        out_shape=jax.ShapeDtypeStruct((VOCAB, 128), jnp.float32),
    )(t416)


def _make_sc_gather(total_rows: int):
    num_cores, num_subcores = 2, 16
    nw = num_cores * num_subcores
    rows_per_w = total_rows // nw
    n_idx_rows = rows_per_w // 128
    GPER = 8
    NGRP = n_idx_rows // GPER
    GROWS = GPER * 128

    mesh = plsc.VectorSubcoreMesh(core_axis_name="c", subcore_axis_name="s",
                                  num_cores=num_cores,
                                  num_subcores=num_subcores)

    @functools.partial(
        pl.kernel,
        mesh=mesh,
        out_type=jax.ShapeDtypeStruct((total_rows, EMB_DIM), jnp.float32),
        scratch_types=[
            pltpu.VMEM((n_idx_rows, 128), jnp.int32),
            pltpu.VMEM((GROWS, EMB_DIM), jnp.float32),
            pltpu.SemaphoreType.DMA,
        ],
        compiler_params=pltpu.CompilerParams(use_tc_tiling_on_sc=False),
    )
    def gather_kernel(table_hbm, idx_hbm, out_hbm, idx_v, rows_v, sem):
        wid = lax.axis_index("s") * num_cores + lax.axis_index("c")
        idx_base = wid * n_idx_rows
        out_base = wid * rows_per_w
        pltpu.sync_copy(idx_hbm.at[pl.ds(idx_base, n_idx_rows)], idx_v)

        def group(g):
            handles = []
            for j in range(GPER):
                handles.append(pltpu.async_copy(
                    table_hbm.at[idx_v.at[g * GPER + j]],
                    rows_v.at[pl.ds(j * 128, 128)],
                    sem))
            for h in handles:
                h.wait()
            pltpu.sync_copy(rows_v, out_hbm.at[pl.ds(out_base + g * GROWS, GROWS)])

        lax.fori_loop(0, NGRP, lambda g, _: (group(g), 0)[1], 0)

    return gather_kernel


def _stats_body(nb, x0_ref, x1_ref, x2_ref, x3_ref, o_ref, acc):
    i = pl.program_id(0)

    @pl.when(i == 0)
    def _():
        acc[...] = jnp.zeros_like(acc)

    x = jnp.concatenate(
        [x0_ref[...], x1_ref[...], x2_ref[...], x3_ref[...]], axis=1)
    s = jnp.sum(x, axis=0, keepdims=True)
    q = jnp.sum(x * x, axis=0, keepdims=True)
    acc[...] += jnp.concatenate([s, q], axis=0)

    @pl.when(i == nb - 1)
    def _():
        o_ref[...] = acc[...]


def _column_stats4(xs, blk):
    b = xs[0].shape[0]
    nb = b // blk
    return pl.pallas_call(
        functools.partial(_stats_body, nb),
        grid=(nb,),
        in_specs=[pl.BlockSpec((blk, 128), lambda i: (i, 0))] * NGRP_F,
        out_specs=pl.BlockSpec((2, D_PAD), lambda i: (0, 0)),
        out_shape=jax.ShapeDtypeStruct((2, D_PAD), jnp.float32),
        scratch_shapes=[pltpu.VMEM((2, D_PAD), jnp.float32)],
    )(*xs)


def _layer1_body(nb, inv_b, x0_ref, x1_ref, x2_ref, x3_ref, st_ref, g_ref,
                 b_ref, w_ref, bias_ref, h_ref, ost_ref, acc):
    i = pl.program_id(0)
    mu = st_ref[0:1, :] * inv_b
    var = st_ref[1:2, :] * inv_b - mu * mu
    s = g_ref[...] * lax.rsqrt(var + EPS)
    t = b_ref[...] - mu * s
    x = jnp.concatenate(
        [x0_ref[...], x1_ref[...], x2_ref[...], x3_ref[...]], axis=1)
    xn = x * s + t
    h = jnp.dot(xn, w_ref[...], preferred_element_type=jnp.float32)
    h = jnp.maximum(h + bias_ref[...], 0.0)
    h_ref[...] = h

    @pl.when(i == 0)
    def _():
        acc[...] = jnp.zeros_like(acc)

    hs = jnp.sum(h, axis=0, keepdims=True)
    hq = jnp.sum(h * h, axis=0, keepdims=True)
    acc[...] += jnp.concatenate([hs, hq], axis=0)

    @pl.when(i == nb - 1)
    def _():
        ost_ref[...] = acc[...]


def _layer1(xs, stats, g, b, w, bias, blk):
    bsz = xs[0].shape[0]
    dout = w.shape[1]
    nb = bsz // blk
    return pl.pallas_call(
        functools.partial(_layer1_body, nb, 1.0 / bsz),
        grid=(nb,),
        in_specs=[pl.BlockSpec((blk, 128), lambda i: (i, 0))] * NGRP_F + [
            pl.BlockSpec((2, D_PAD), lambda i: (0, 0)),
            pl.BlockSpec((1, D_PAD), lambda i: (0, 0)),
            pl.BlockSpec((1, D_PAD), lambda i: (0, 0)),
            pl.BlockSpec((D_PAD, dout), lambda i: (0, 0)),
            pl.BlockSpec((1, dout), lambda i: (0, 0)),
        ],
        out_specs=[
            pl.BlockSpec((blk, dout), lambda i: (i, 0)),
            pl.BlockSpec((2, dout), lambda i: (0, 0)),
        ],
        out_shape=[
            jax.ShapeDtypeStruct((bsz, dout), jnp.float32),
            jax.ShapeDtypeStruct((2, dout), jnp.float32),
        ],
        scratch_shapes=[pltpu.VMEM((2, dout), jnp.float32)],
    )(*xs, stats, g, b, w, bias)


def _layer_body(nb, inv_b, x_ref, st_ref, g_ref, b_ref, w_ref, bias_ref,
                h_ref, ost_ref, acc):
    i = pl.program_id(0)
    mu = st_ref[0:1, :] * inv_b
    var = st_ref[1:2, :] * inv_b - mu * mu
    s = g_ref[...] * lax.rsqrt(var + EPS)
    t = b_ref[...] - mu * s
    xn = x_ref[...] * s + t
    h = jnp.dot(xn, w_ref[...], preferred_element_type=jnp.float32)
    h = jnp.maximum(h + bias_ref[...], 0.0)
    h_ref[...] = h

    @pl.when(i == 0)
    def _():
        acc[...] = jnp.zeros_like(acc)

    hs = jnp.sum(h, axis=0, keepdims=True)
    hq = jnp.sum(h * h, axis=0, keepdims=True)
    acc[...] += jnp.concatenate([hs, hq], axis=0)

    @pl.when(i == nb - 1)
    def _():
        ost_ref[...] = acc[...]


def _norm_layer(x, stats, g, b, w, bias, blk):
    bsz, din = x.shape
    dout = w.shape[1]
    nb = bsz // blk
    return pl.pallas_call(
        functools.partial(_layer_body, nb, 1.0 / bsz),
        grid=(nb,),
        in_specs=[
            pl.BlockSpec((blk, din), lambda i: (i, 0)),
            pl.BlockSpec((2, din), lambda i: (0, 0)),
            pl.BlockSpec((1, din), lambda i: (0, 0)),
            pl.BlockSpec((1, din), lambda i: (0, 0)),
            pl.BlockSpec((din, dout), lambda i: (0, 0)),
            pl.BlockSpec((1, dout), lambda i: (0, 0)),
        ],
        out_specs=[
            pl.BlockSpec((blk, dout), lambda i: (i, 0)),
            pl.BlockSpec((2, dout), lambda i: (0, 0)),
        ],
        out_shape=[
            jax.ShapeDtypeStruct((bsz, dout), jnp.float32),
            jax.ShapeDtypeStruct((2, dout), jnp.float32),
        ],
        scratch_shapes=[pltpu.VMEM((2, dout), jnp.float32)],
    )(x, stats, g.reshape(1, din), b.reshape(1, din), w, bias.reshape(1, dout))


def _final_body(inv_b, x_ref, st_ref, g_ref, b_ref, w_ref, bias_ref, o_ref):
    mu = st_ref[0:1, :] * inv_b
    var = st_ref[1:2, :] * inv_b - mu * mu
    s = g_ref[...] * lax.rsqrt(var + EPS)
    t = b_ref[...] - mu * s
    xn = x_ref[...] * s + t
    z = jnp.sum(xn * w_ref[...], axis=1, keepdims=True) + bias_ref[0, 0]
    o_ref[...] = jax.nn.sigmoid(z)


def _final_layer(x, stats, g, b, w3, b3, blk):
    bsz, din = x.shape
    nb = bsz // blk
    return pl.pallas_call(
        functools.partial(_final_body, 1.0 / bsz),
        grid=(nb,),
        in_specs=[
            pl.BlockSpec((blk, din), lambda i: (i, 0)),
            pl.BlockSpec((2, din), lambda i: (0, 0)),
            pl.BlockSpec((1, din), lambda i: (0, 0)),
            pl.BlockSpec((1, din), lambda i: (0, 0)),
            pl.BlockSpec((1, din), lambda i: (0, 0)),
            pl.BlockSpec((1, 1), lambda i: (0, 0)),
        ],
        out_specs=pl.BlockSpec((blk, 1), lambda i: (i, 0)),
        out_shape=jax.ShapeDtypeStruct((bsz, 1), jnp.float32),
    )(x, stats, g.reshape(1, din), b.reshape(1, din),
      w3.reshape(1, din), b3.reshape(1, 1))


def kernel(x_cat, tables, W1, b1, W2, b2, W3, b3,
           bn0_g, bn0_b, bn1_g, bn1_b, bn2_g, bn2_b):
    bsz = x_cat.shape[0]

    t416 = tables.transpose(0, 2, 1).reshape(NUM_FIELDS * EMB_DIM, VOCAB)

    xc = x_cat.astype(jnp.int32)
    ndum = NGRP_F * 8 - NUM_FIELDS
    dum = jnp.broadcast_to(
        (jnp.arange(bsz, dtype=jnp.int32) * 7919) % VOCAB, (ndum, bsz)).T
    xc_pad = jnp.concatenate([xc, dum], axis=1)
    fl = jnp.arange(NGRP_F * 8, dtype=jnp.int32)[None, :] % 8
    idx_all = xc_pad * 8 + fl

    gather = _make_sc_gather(bsz * 8)
    xs = []
    for g in range(NGRP_F):
        tg = _table_group(t416, g).reshape(VOCAB * 8, EMB_DIM)
        idx_g = idx_all[:, 8 * g:8 * (g + 1)].reshape(bsz * 8 // 128, 128)
        rows = gather(tg, idx_g)
        xs.append(rows.reshape(bsz, 128))

    dpadc = D_PAD - NUM_FIELDS * EMB_DIM
    g0 = jnp.concatenate([bn0_g, jnp.zeros((dpadc,), jnp.float32)])
    b0 = jnp.concatenate([bn0_b, jnp.zeros((dpadc,), jnp.float32)])
    w1 = jnp.concatenate(
        [W1, jnp.zeros((dpadc, W1.shape[1]), jnp.float32)], axis=0)

    blk = 2048
    st0 = _column_stats4(xs, blk)
    h1, st1 = _layer1(xs, st0, g0.reshape(1, D_PAD), b0.reshape(1, D_PAD),
                      w1, b1.reshape(1, -1), blk)
    h2, st2 = _norm_layer(h1, st1, bn1_g, bn1_b, W2, b2, blk)
    out = _final_layer(h2, st2, bn2_g, bn2_b, W3, b3, blk)
    return out.reshape(bsz)
